# Initial kernel scaffold; baseline (speedup 1.0000x reference)
import jax
import jax.numpy as jnp
from jax.experimental import pallas as pl


def kernel(x, edge_index, W, gamma, beta):
    raise NotImplementedError("write your pallas kernel here")


# speedup vs baseline: 12.6124x; 12.6124x over previous
import functools

import jax
import jax.numpy as jnp
from jax import lax
from jax.experimental import pallas as pl
from jax.experimental.pallas import tpu as pltpu
from jax.experimental.pallas import tpu_sc as plsc

B, C, N, K = 1, 128, 10000, 32
COUT = 128
NK = N * K

NC, NS = 2, 16
NW = NC * NS
NP = -(-N // NW)
NPAD = NW * NP

BN = 1000


def _mm_body(xst_ref, wt_ref, ut_ref, vt_ref):
    xsb = xst_ref[...]
    wt = wt_ref[...]
    at = wt[:C, :] - wt[C:, :]
    ut_ref[...] = jnp.dot(xsb, at, preferred_element_type=jnp.float32)
    vt_ref[...] = jnp.dot(xsb, wt[C:, :], preferred_element_type=jnp.float32)


def _sc_body(ut_hbm, vt_hbm, ii_hbm, jj_hbm,
             maxo_hbm, mino_hbm, sums_hbm, sumsqs_hbm,
             ii_v, jj_v, bufi_v, bufj_v, mx_v, mn_v, s_v, ss_v,
             semi, semj):
    wid = lax.axis_index("s") * NC + lax.axis_index("c")
    base = wid * NP
    cnt = jnp.minimum(NP, N - base)

    pltpu.sync_copy(ii_hbm.at[wid], ii_v)
    pltpu.sync_copy(jj_hbm.at[wid], jj_v)

    zero = jnp.zeros((16,), jnp.float32)
    for c in range(C // 16):
        sl = pl.ds(c * 16, 16)
        s_v[sl] = zero
        ss_v[sl] = zero

    def node_body(t, carry):
        cp_i = pltpu.async_copy(ut_hbm.at[ii_v.at[t]], bufi_v, semi)
        cp_j = pltpu.async_copy(vt_hbm.at[jj_v.at[t]], bufj_v, semj)
        cp_i.wait()
        cp_j.wait()
        for c in range(C // 16):
            sl = pl.ds(c * 16, 16)
            mx = jnp.full((16,), -jnp.inf, jnp.float32)
            mn = jnp.full((16,), jnp.inf, jnp.float32)
            s = s_v[sl]
            ss = ss_v[sl]
            for k in range(K):
                y = bufi_v[k, sl] + bufj_v[k, sl]
                mx = jnp.maximum(mx, y)
                mn = jnp.minimum(mn, y)
                s = s + y
                ss = ss + y * y
            mx_v[sl] = mx
            mn_v[sl] = mn
            s_v[sl] = s
            ss_v[sl] = ss
        n = base + t
        pltpu.sync_copy(mx_v, maxo_hbm.at[n])
        pltpu.sync_copy(mn_v, mino_hbm.at[n])
        return carry

    lax.fori_loop(0, cnt, node_body, 0)
    pltpu.sync_copy(s_v, sums_hbm.at[wid])
    pltpu.sync_copy(ss_v, sumsqs_hbm.at[wid])


def _fin_body(mx_ref, mn_ref, sums_ref, sumsqs_ref, g_ref, b_ref, out_ref):
    s = jnp.sum(sums_ref[...], axis=0, keepdims=True)
    ss = jnp.sum(sumsqs_ref[...], axis=0, keepdims=True)
    mean = s / NK
    var = ss / NK - mean * mean
    a = g_ref[...] * lax.rsqrt(var + 1e-5)
    b = b_ref[...] - a * mean
    sel = jnp.where(a >= 0, mx_ref[...], mn_ref[...])
    res = jnp.maximum(sel * a + b, 0.0)
    out_ref[...] = res.T


def kernel(x, edge_index, W, gamma, beta):
    f32 = jnp.float32
    xst = x.reshape(C, N).T
    wt = W.T

    ut, vt = pl.pallas_call(
        _mm_body,
        grid=(N // BN,),
        in_specs=[
            pl.BlockSpec((BN, C), lambda i: (i, 0)),
            pl.BlockSpec((2 * C, COUT), lambda i: (0, 0)),
        ],
        out_specs=[
            pl.BlockSpec((BN, COUT), lambda i: (i, 0)),
            pl.BlockSpec((BN, COUT), lambda i: (i, 0)),
        ],
        out_shape=[
            jax.ShapeDtypeStruct((N, COUT), f32),
            jax.ShapeDtypeStruct((N, COUT), f32),
        ],
    )(xst, wt)

    pad = NPAD - N
    ii = jnp.pad(edge_index[1].reshape(N, K).astype(jnp.int32),
                 ((0, pad), (0, 0))).reshape(NW, NP, K)
    jj = jnp.pad(edge_index[0].reshape(N, K).astype(jnp.int32),
                 ((0, pad), (0, 0))).reshape(NW, NP, K)

    mesh = plsc.VectorSubcoreMesh(core_axis_name="c", subcore_axis_name="s")
    sc_fn = functools.partial(
        pl.kernel,
        mesh=mesh,
        out_type=[
            jax.ShapeDtypeStruct((N, COUT), f32),
            jax.ShapeDtypeStruct((N, COUT), f32),
            jax.ShapeDtypeStruct((NW, COUT), f32),
            jax.ShapeDtypeStruct((NW, COUT), f32),
        ],
        scratch_types=[
            pltpu.VMEM((NP, K), jnp.int32),
            pltpu.VMEM((NP, K), jnp.int32),
            pltpu.VMEM((K, COUT), f32),
            pltpu.VMEM((K, COUT), f32),
            pltpu.VMEM((COUT,), f32),
            pltpu.VMEM((COUT,), f32),
            pltpu.VMEM((COUT,), f32),
            pltpu.VMEM((COUT,), f32),
            pltpu.SemaphoreType.DMA,
            pltpu.SemaphoreType.DMA,
        ],
    )(_sc_body)
    mx, mn, sums, sumsqs = sc_fn(ut, vt, ii, jj)

    out = pl.pallas_call(
        _fin_body,
        out_shape=jax.ShapeDtypeStruct((COUT, N), f32),
    )(mx, mn, sums, sumsqs, gamma.reshape(1, COUT), beta.reshape(1, COUT))

    return out.reshape(B, COUT, N, 1)


# speedup vs baseline: 15.7882x; 1.2518x over previous
import functools

import jax
import jax.numpy as jnp
from jax import lax
from jax.experimental import pallas as pl
from jax.experimental.pallas import tpu as pltpu
from jax.experimental.pallas import tpu_sc as plsc

B, C, N, K = 1, 128, 10000, 32
COUT = 128
NK = N * K

NC, NS = 2, 16
NW = NC * NS
NP = 314
NPAD = NW * NP
NT = NPAD
NCH = C // 16


def _mm_body(xst_ref, wt_ref, tab_ref):
    xsb = xst_ref[...]
    wt = wt_ref[...]
    at = wt[:C, :] - wt[C:, :]
    tab_ref[:NT, :] = jnp.dot(xsb, at, preferred_element_type=jnp.float32)
    tab_ref[NT:, :] = jnp.dot(xsb, wt[C:, :], preferred_element_type=jnp.float32)


def _sc_body(tab_hbm, idx_hbm,
             out2_hbm, sums_hbm, sumsqs_hbm,
             idx_v, buf_v, outb_v, s_v, ss_v,
             gsem0, gsem1, osem0, osem1):
    wid = lax.axis_index("s") * NC + lax.axis_index("c")
    base = wid * NP
    gsem = (gsem0, gsem1)
    osem = (osem0, osem1)

    pltpu.sync_copy(idx_hbm.at[wid], idx_v)

    pltpu.async_copy(tab_hbm.at[idx_v.at[0]], buf_v.at[0], gsem[0])
    pltpu.async_copy(tab_hbm.at[idx_v.at[1]], buf_v.at[1], gsem[1])

    zero = jnp.zeros((16,), jnp.float32)
    init = tuple(zero for _ in range(2 * NCH))

    def pair_body(g, carry):
        acc = list(carry)
        for b in range(2):
            t = g * 2 + b
            n = base + t
            pltpu.make_async_copy(
                tab_hbm.at[idx_v.at[t]], buf_v.at[b], gsem[b]).wait()
            @pl.when(t >= 2)
            def _():
                pltpu.make_async_copy(
                    outb_v.at[b], out2_hbm.at[n - 2], osem[b]).wait()
            for c in range(NCH):
                sl = pl.ds(c * 16, 16)
                mx = jnp.full((16,), -jnp.inf, jnp.float32)
                mn = jnp.full((16,), jnp.inf, jnp.float32)
                s = acc[2 * c]
                ss = acc[2 * c + 1]
                for k in range(K):
                    y = buf_v[b, k, sl] + buf_v[b, K + k, sl]
                    mx = jnp.maximum(mx, y)
                    mn = jnp.minimum(mn, y)
                    s = s + y
                    ss = ss + y * y
                acc[2 * c] = s
                acc[2 * c + 1] = ss
                outb_v[b, 0, sl] = mx
                outb_v[b, 1, sl] = mn
            pltpu.async_copy(outb_v.at[b], out2_hbm.at[n], osem[b])
            @pl.when(t + 2 < NP)
            def _():
                pltpu.async_copy(
                    tab_hbm.at[idx_v.at[t + 2]], buf_v.at[b], gsem[b])
        return tuple(acc)

    acc = lax.fori_loop(0, NP // 2, pair_body, init)

    for b in range(2):
        pltpu.make_async_copy(
            outb_v.at[b], out2_hbm.at[base + NP - 2 + b], osem[b]).wait()

    for c in range(NCH):
        sl = pl.ds(c * 16, 16)
        s_v[sl] = acc[2 * c]
        ss_v[sl] = acc[2 * c + 1]
    pltpu.sync_copy(s_v, sums_hbm.at[wid])
    pltpu.sync_copy(ss_v, sumsqs_hbm.at[wid])


def _fin_body(mxmn_ref, sums_ref, sumsqs_ref, g_ref, b_ref, out_ref):
    s = jnp.sum(sums_ref[...], axis=0, keepdims=True)
    ss = jnp.sum(sumsqs_ref[...], axis=0, keepdims=True)
    mean = s / NK
    var = ss / NK - mean * mean
    a = g_ref[...] * lax.rsqrt(var + 1e-5)
    b = b_ref[...] - a * mean
    sel = jnp.where(a >= 0, mxmn_ref[:, 0, :], mxmn_ref[:, 1, :])
    res = jnp.maximum(sel * a + b, 0.0)
    out_ref[...] = res.T


def kernel(x, edge_index, W, gamma, beta):
    f32 = jnp.float32
    xst = jnp.pad(x.reshape(C, N).T, ((0, NT - N), (0, 0)))
    wt = W.T

    tab = pl.pallas_call(
        _mm_body,
        out_shape=jax.ShapeDtypeStruct((2 * NT, COUT), f32),
    )(xst, wt)

    ii = edge_index[1].reshape(N, K).astype(jnp.int32)
    jj = edge_index[0].reshape(N, K).astype(jnp.int32)
    idx = jnp.concatenate([ii, jj + NT], axis=1)
    idx = jnp.pad(idx, ((0, NPAD - N), (0, 0)), constant_values=N)
    idx = idx.reshape(NW, NP, 2 * K)

    mesh = plsc.VectorSubcoreMesh(core_axis_name="c", subcore_axis_name="s")
    sc_fn = functools.partial(
        pl.kernel,
        mesh=mesh,
        out_type=[
            jax.ShapeDtypeStruct((NPAD, 2, COUT), f32),
            jax.ShapeDtypeStruct((NW, COUT), f32),
            jax.ShapeDtypeStruct((NW, COUT), f32),
        ],
        scratch_types=[
            pltpu.VMEM((NP, 2 * K), jnp.int32),
            pltpu.VMEM((2, 2 * K, COUT), f32),
            pltpu.VMEM((2, 2, COUT), f32),
            pltpu.VMEM((COUT,), f32),
            pltpu.VMEM((COUT,), f32),
            pltpu.SemaphoreType.DMA,
            pltpu.SemaphoreType.DMA,
            pltpu.SemaphoreType.DMA,
            pltpu.SemaphoreType.DMA,
        ],
    )(_sc_body)
    mxmn, sums, sumsqs = sc_fn(tab, idx)

    out = pl.pallas_call(
        _fin_body,
        out_shape=jax.ShapeDtypeStruct((COUT, NPAD), f32),
    )(mxmn, sums, sumsqs, gamma.reshape(1, COUT), beta.reshape(1, COUT))

    return out[:, :N].reshape(B, COUT, N, 1)
